# baseline (speedup 1.0000x reference)
import jax
import jax.numpy as jnp
from jax.experimental import pallas as pl

N = 10000


def _copy_body(x_ref, o_ref):
    o_ref[...] = x_ref[...]


def _graph_conv(h, src, dst, norm_src, norm_dst, W, b, act):
    h = h * norm_src[:, None]
    m = jax.ops.segment_sum(h[src], dst, num_segments=N)
    m = m * norm_dst[:, None]
    out = m @ W + b
    if act:
        out = jax.nn.relu(out)
    return out


def kernel(x, edge_index, W0, b0, W1, b1, W2, b2):
    E = edge_index.shape[1]
    src = edge_index[0]
    dst = edge_index[1]
    ones = jnp.ones((E,), dtype=jnp.float32)
    deg_out = jax.ops.segment_sum(ones, src, num_segments=N)
    deg_in = jax.ops.segment_sum(ones, dst, num_segments=N)
    norm_src = jnp.power(jnp.clip(deg_out, 1.0, None), -0.5)
    norm_dst = jnp.power(jnp.clip(deg_in, 1.0, None), -0.5)
    h = _graph_conv(x, src, dst, norm_src, norm_dst, W0, b0, True)
    h = _graph_conv(h, src, dst, norm_src, norm_dst, W1, b1, True)
    h = _graph_conv(h, src, dst, norm_src, norm_dst, W2, b2, False)
    h = pl.pallas_call(
        _copy_body,
        out_shape=jax.ShapeDtypeStruct(h.shape, h.dtype),
    )(h)
    return h


# speedup vs baseline: 4.0011x; 4.0011x over previous
import functools

import jax
import jax.numpy as jnp
from jax import lax
from jax.experimental import pallas as pl
from jax.experimental.pallas import tpu as pltpu
from jax.experimental.pallas import tpu_sc as plsc

N = 10000
E = 320000
NP = 10112
NZ = NP // 16
NC = 2
NS = 16
K = 80

_MESH = plsc.VectorSubcoreMesh(core_axis_name="c", subcore_axis_name="s")


def _degrees(src, dst, zeros128, ones128):
    epw = E // NS
    nchunk = epw // K

    @functools.partial(
        pl.kernel,
        out_type=jax.ShapeDtypeStruct((NC, NP, 128), jnp.float32),
        mesh=_MESH,
        scratch_types=[
            pltpu.VMEM((K,), jnp.int32),
            pltpu.VMEM((K, 128), jnp.float32),
            pltpu.VMEM_SHARED((NP, 128), jnp.float32),
            pltpu.SemaphoreType.DMA,
        ],
    )
    def deg_kernel(src_hbm, dst_hbm, z_hbm, o_hbm, out_hbm, idx_v, ones_v, acc, sem):
        c = lax.axis_index("c")
        s = lax.axis_index("s")
        pltpu.sync_copy(o_hbm, ones_v)
        pltpu.sync_copy(z_hbm.at[pl.ds(s * NZ, NZ)], acc.at[pl.ds(s * NZ, NZ)])
        plsc.subcore_barrier()

        def hist(e_hbm):
            @pl.loop(0, nchunk)
            def _(i):
                off = s * epw + i * K
                pltpu.sync_copy(e_hbm.at[pl.ds(off, K)], idx_v)
                pltpu.sync_copy(ones_v, acc.at[idx_v], add=True)

        @pl.when(c == 0)
        def _():
            hist(src_hbm)

        @pl.when(c == 1)
        def _():
            hist(dst_hbm)

        plsc.subcore_barrier()
        pltpu.async_copy(
            acc.at[pl.ds(s * NZ, NZ)], out_hbm.at[c].at[pl.ds(s * NZ, NZ)], sem
        ).wait()

    return deg_kernel(src, dst, zeros128, ones128)


def _propagate(g, src, dst, zeros, width):
    epw = E // (NC * NS)
    nchunk = epw // K

    @functools.partial(
        pl.kernel,
        out_type=jax.ShapeDtypeStruct((NC, NP, width), jnp.float32),
        mesh=_MESH,
        scratch_types=[
            pltpu.VMEM((K,), jnp.int32),
            pltpu.VMEM((K,), jnp.int32),
            pltpu.VMEM((K, width), jnp.float32),
            pltpu.VMEM_SHARED((NP, width), jnp.float32),
            pltpu.SemaphoreType.DMA,
        ],
    )
    def prop_kernel(g_hbm, src_hbm, dst_hbm, z_hbm, out_hbm,
                    sidx, didx, rows, acc, sem):
        c = lax.axis_index("c")
        s = lax.axis_index("s")
        pltpu.sync_copy(z_hbm.at[pl.ds(s * NZ, NZ)], acc.at[pl.ds(s * NZ, NZ)])
        plsc.subcore_barrier()

        base = (c * NS + s) * epw

        @pl.loop(0, nchunk)
        def _(i):
            off = base + i * K
            pltpu.sync_copy(src_hbm.at[pl.ds(off, K)], sidx)
            pltpu.sync_copy(dst_hbm.at[pl.ds(off, K)], didx)
            pltpu.async_copy(g_hbm.at[sidx], rows, sem).wait()
            pltpu.sync_copy(rows, acc.at[didx], add=True)

        plsc.subcore_barrier()
        pltpu.async_copy(
            acc.at[pl.ds(s * NZ, NZ)], out_hbm.at[c].at[pl.ds(s * NZ, NZ)], sem
        ).wait()

    return prop_kernel(g, src, dst, zeros)


_BLK = 632


def _norm(deg_blk):
    return lax.rsqrt(jnp.clip(deg_blk[:, :1], 1.0, None))


def _matmul(x, W):
    def body(x_ref, w_ref, o_ref):
        o_ref[...] = jnp.dot(x_ref[...], w_ref[...], precision=lax.Precision.HIGHEST,
                             preferred_element_type=jnp.float32)

    m, k = x.shape
    n = W.shape[1]
    return pl.pallas_call(
        body,
        grid=(m // _BLK,),
        in_specs=[
            pl.BlockSpec((_BLK, k), lambda i: (i, 0)),
            pl.BlockSpec((k, n), lambda i: (0, 0)),
        ],
        out_specs=pl.BlockSpec((_BLK, n), lambda i: (i, 0)),
        out_shape=jax.ShapeDtypeStruct((m, n), jnp.float32),
    )(x, W)


def _scale_rows(z, deg):
    def body(z_ref, d_ref, o_ref):
        o_ref[...] = z_ref[...] * _norm(d_ref[...])

    m, n = z.shape
    return pl.pallas_call(
        body,
        grid=(m // _BLK,),
        in_specs=[
            pl.BlockSpec((_BLK, n), lambda i: (i, 0)),
            pl.BlockSpec((_BLK, 128), lambda i: (i, 0)),
        ],
        out_specs=pl.BlockSpec((_BLK, n), lambda i: (i, 0)),
        out_shape=jax.ShapeDtypeStruct((m, n), jnp.float32),
    )(z, deg)


def _fused_layer(ppart, deg_in, deg_out, b, W):
    def body(p_ref, di_ref, do_ref, b_ref, w_ref, o_ref):
        p = p_ref[0] + p_ref[1]
        h = jax.nn.relu(p * _norm(di_ref[...]) + b_ref[...])
        o_ref[...] = jnp.dot(h, w_ref[...], precision=lax.Precision.HIGHEST,
                             preferred_element_type=jnp.float32) * _norm(do_ref[...])

    _, m, k = ppart.shape
    n = W.shape[1]
    return pl.pallas_call(
        body,
        grid=(m // _BLK,),
        in_specs=[
            pl.BlockSpec((NC, _BLK, k), lambda i: (0, i, 0)),
            pl.BlockSpec((_BLK, 128), lambda i: (i, 0)),
            pl.BlockSpec((_BLK, 128), lambda i: (i, 0)),
            pl.BlockSpec((1, k), lambda i: (0, 0)),
            pl.BlockSpec((k, n), lambda i: (0, 0)),
        ],
        out_specs=pl.BlockSpec((_BLK, n), lambda i: (i, 0)),
        out_shape=jax.ShapeDtypeStruct((m, n), jnp.float32),
    )(ppart, deg_in, deg_out, b, W)


def _act_scale(ppart, deg_in, deg_out, b):
    def body(p_ref, di_ref, do_ref, b_ref, o_ref):
        p = p_ref[0] + p_ref[1]
        h = jax.nn.relu(p * _norm(di_ref[...]) + b_ref[...])
        o_ref[...] = h * _norm(do_ref[...])

    _, m, k = ppart.shape
    return pl.pallas_call(
        body,
        grid=(m // _BLK,),
        in_specs=[
            pl.BlockSpec((NC, _BLK, k), lambda i: (0, i, 0)),
            pl.BlockSpec((_BLK, 128), lambda i: (i, 0)),
            pl.BlockSpec((_BLK, 128), lambda i: (i, 0)),
            pl.BlockSpec((1, k), lambda i: (0, 0)),
        ],
        out_specs=pl.BlockSpec((_BLK, k), lambda i: (i, 0)),
        out_shape=jax.ShapeDtypeStruct((m, k), jnp.float32),
    )(ppart, deg_in, deg_out, b)


def _final_layer(ppart, deg_in, b, W):
    def body(p_ref, di_ref, b_ref, w_ref, o_ref):
        p = p_ref[0] + p_ref[1]
        m = p * _norm(di_ref[...])
        o_ref[...] = jnp.dot(m, w_ref[...], precision=lax.Precision.HIGHEST,
                             preferred_element_type=jnp.float32) + b_ref[...]

    _, m, k = ppart.shape
    n = W.shape[1]
    return pl.pallas_call(
        body,
        grid=(m // _BLK,),
        in_specs=[
            pl.BlockSpec((NC, _BLK, k), lambda i: (0, i, 0)),
            pl.BlockSpec((_BLK, 128), lambda i: (i, 0)),
            pl.BlockSpec((1, n), lambda i: (0, 0)),
            pl.BlockSpec((k, n), lambda i: (0, 0)),
        ],
        out_specs=pl.BlockSpec((_BLK, n), lambda i: (i, 0)),
        out_shape=jax.ShapeDtypeStruct((m, n), jnp.float32),
    )(ppart, deg_in, b, W)


def kernel(x, edge_index, W0, b0, W1, b1, W2, b2):
    assert x.shape == (N, 128) and edge_index.shape == (2, E)
    src = edge_index[0].astype(jnp.int32)
    dst = edge_index[1].astype(jnp.int32)

    C = W2.shape[1]
    W2p = jnp.pad(W2, ((0, 0), (0, 64 - C)))
    b2p = jnp.pad(b2, (0, 64 - C))

    xp = jnp.pad(x, ((0, NP - N), (0, 0)))

    zeros128 = jnp.zeros((NP, 128), jnp.float32)
    ones128 = jnp.ones((K, 128), jnp.float32)

    degs = _degrees(src, dst, zeros128, ones128)
    z0 = _matmul(xp, W0)
    deg_out = degs[0]
    deg_in = degs[1]

    g0 = _scale_rows(z0, deg_out)
    p0 = _propagate(g0, src, dst, zeros128, 128)
    g1 = _fused_layer(p0, deg_in, deg_out, b0.reshape(1, -1), W1)
    p1 = _propagate(g1, src, dst, zeros128, 128)
    g2 = _act_scale(p1, deg_in, deg_out, b1.reshape(1, -1))
    p2 = _propagate(g2, src, dst, zeros128, 128)
    out = _final_layer(p2, deg_in, b2p.reshape(1, -1), W2p)
    return out[:N, :C]


# speedup vs baseline: 7.9818x; 1.9949x over previous
import functools

import jax
import jax.numpy as jnp
from jax import lax
from jax.experimental import pallas as pl
from jax.experimental.pallas import tpu as pltpu
from jax.experimental.pallas import tpu_sc as plsc

N = 10000
E = 320000
NP = 10112
NZ = NP // 16
NC = 2
NS = 16
KP = 125
ROWS = E // KP

_MESH = plsc.VectorSubcoreMesh(core_axis_name="c", subcore_axis_name="s")


def _degrees(ei2d, zeros128, ones128):
    rpw = ROWS // NS
    FIRE = 8

    @functools.partial(
        pl.kernel,
        out_type=jax.ShapeDtypeStruct((NC, NP, 128), jnp.float32),
        mesh=_MESH,
        scratch_types=[
            pltpu.VMEM((rpw, KP), jnp.int32),
            pltpu.VMEM((KP, 128), jnp.float32),
            pltpu.VMEM_SHARED((NP, 128), jnp.float32),
            pltpu.SemaphoreType.DMA,
        ],
    )
    def deg_kernel(ei_hbm, z_hbm, o_hbm, out_hbm, idx_v, ones_v, acc, sem):
        c = lax.axis_index("c")
        s = lax.axis_index("s")
        pltpu.sync_copy(o_hbm, ones_v)
        pltpu.sync_copy(z_hbm.at[pl.ds(s * NZ, NZ)], acc.at[pl.ds(s * NZ, NZ)])
        pltpu.sync_copy(ei_hbm.at[pl.ds(c * ROWS + s * rpw, rpw)], idx_v)
        plsc.subcore_barrier()

        @pl.loop(0, rpw, step=FIRE)
        def _(i):
            for b in range(FIRE):
                pltpu.async_copy(ones_v, acc.at[idx_v.at[i + b]], sem, add=True)
            for b in range(FIRE):
                pltpu.make_async_copy(o_hbm, ones_v, sem).wait()

        plsc.subcore_barrier()
        pltpu.async_copy(
            acc.at[pl.ds(s * NZ, NZ)], out_hbm.at[c].at[pl.ds(s * NZ, NZ)], sem
        ).wait()

    return deg_kernel(ei2d, zeros128, ones128)


def _propagate(g, src2d, dst2d, zeros128):
    rpw = ROWS // (NC * NS)
    rps = 8
    SEG = rpw // rps

    @functools.partial(
        pl.kernel,
        out_type=jax.ShapeDtypeStruct((NC, NP, 128), jnp.float32),
        mesh=_MESH,
        scratch_types=[
            pltpu.VMEM((rps, KP), jnp.int32),
            pltpu.VMEM((rps, KP), jnp.int32),
            pltpu.VMEM((KP, 128), jnp.float32),
            pltpu.VMEM((KP, 128), jnp.float32),
            pltpu.VMEM_SHARED((NP, 128), jnp.float32),
            pltpu.SemaphoreType.DMA,
            pltpu.SemaphoreType.DMA,
        ],
    )
    def prop_kernel(g_hbm, s_hbm, d_hbm, z_hbm, out_hbm,
                    sidx, didx, rows0, rows1, acc, sem0, sem1):
        c = lax.axis_index("c")
        s = lax.axis_index("s")
        r0 = (c * NS + s) * rpw
        pltpu.sync_copy(z_hbm.at[pl.ds(s * NZ, NZ)], acc.at[pl.ds(s * NZ, NZ)])
        plsc.subcore_barrier()

        rows = (rows0, rows1)
        sems = (sem0, sem1)

        @pl.loop(0, SEG)
        def _(seg):
            rseg = r0 + seg * rps
            pltpu.sync_copy(s_hbm.at[pl.ds(rseg, rps)], sidx)
            pltpu.sync_copy(d_hbm.at[pl.ds(rseg, rps)], didx)
            pltpu.async_copy(g_hbm.at[sidx.at[0]], rows0, sem0)
            pltpu.async_copy(g_hbm.at[sidx.at[1]], rows1, sem1)

            @pl.loop(0, rps, step=2)
            def _(i):
                for b in range(2):
                    j = i + b
                    pltpu.make_async_copy(g_hbm.at[sidx.at[j]], rows[b], sems[b]).wait()
                    pltpu.sync_copy(rows[b], acc.at[didx.at[j]], add=True)

                    @pl.when(j + 2 < rps)
                    def _():
                        pltpu.async_copy(g_hbm.at[sidx.at[j + 2]], rows[b], sems[b])

        plsc.subcore_barrier()
        pltpu.async_copy(
            acc.at[pl.ds(s * NZ, NZ)], out_hbm.at[c].at[pl.ds(s * NZ, NZ)], sem0
        ).wait()

    return prop_kernel(g, src2d, dst2d, zeros128)


_BLK = 632


def _norm(deg_blk):
    return lax.rsqrt(jnp.clip(deg_blk[:, :1], 1.0, None))


def _matmul(x, W):
    def body(x_ref, w_ref, o_ref):
        o_ref[...] = jnp.dot(x_ref[...], w_ref[...], precision=lax.Precision.HIGHEST,
                             preferred_element_type=jnp.float32)

    m, k = x.shape
    n = W.shape[1]
    return pl.pallas_call(
        body,
        grid=(m // _BLK,),
        in_specs=[
            pl.BlockSpec((_BLK, k), lambda i: (i, 0)),
            pl.BlockSpec((k, n), lambda i: (0, 0)),
        ],
        out_specs=pl.BlockSpec((_BLK, n), lambda i: (i, 0)),
        out_shape=jax.ShapeDtypeStruct((m, n), jnp.float32),
    )(x, W)


def _scale_rows(z, deg):
    def body(z_ref, d_ref, o_ref):
        o_ref[...] = z_ref[...] * _norm(d_ref[...])

    m, n = z.shape
    return pl.pallas_call(
        body,
        grid=(m // _BLK,),
        in_specs=[
            pl.BlockSpec((_BLK, n), lambda i: (i, 0)),
            pl.BlockSpec((_BLK, 128), lambda i: (i, 0)),
        ],
        out_specs=pl.BlockSpec((_BLK, n), lambda i: (i, 0)),
        out_shape=jax.ShapeDtypeStruct((m, n), jnp.float32),
    )(z, deg)


def _fused_layer(ppart, deg_in, deg_out, b, W):
    def body(p_ref, di_ref, do_ref, b_ref, w_ref, o_ref):
        p = p_ref[0] + p_ref[1]
        h = jax.nn.relu(p * _norm(di_ref[...]) + b_ref[...])
        o_ref[...] = jnp.dot(h, w_ref[...], precision=lax.Precision.HIGHEST,
                             preferred_element_type=jnp.float32) * _norm(do_ref[...])

    _, m, k = ppart.shape
    n = W.shape[1]
    return pl.pallas_call(
        body,
        grid=(m // _BLK,),
        in_specs=[
            pl.BlockSpec((NC, _BLK, k), lambda i: (0, i, 0)),
            pl.BlockSpec((_BLK, 128), lambda i: (i, 0)),
            pl.BlockSpec((_BLK, 128), lambda i: (i, 0)),
            pl.BlockSpec((1, k), lambda i: (0, 0)),
            pl.BlockSpec((k, n), lambda i: (0, 0)),
        ],
        out_specs=pl.BlockSpec((_BLK, n), lambda i: (i, 0)),
        out_shape=jax.ShapeDtypeStruct((m, n), jnp.float32),
    )(ppart, deg_in, deg_out, b, W)


def _act_scale(ppart, deg_in, deg_out, b):
    def body(p_ref, di_ref, do_ref, b_ref, o_ref):
        p = p_ref[0] + p_ref[1]
        h = jax.nn.relu(p * _norm(di_ref[...]) + b_ref[...])
        o_ref[...] = h * _norm(do_ref[...])

    _, m, k = ppart.shape
    return pl.pallas_call(
        body,
        grid=(m // _BLK,),
        in_specs=[
            pl.BlockSpec((NC, _BLK, k), lambda i: (0, i, 0)),
            pl.BlockSpec((_BLK, 128), lambda i: (i, 0)),
            pl.BlockSpec((_BLK, 128), lambda i: (i, 0)),
            pl.BlockSpec((1, k), lambda i: (0, 0)),
        ],
        out_specs=pl.BlockSpec((_BLK, k), lambda i: (i, 0)),
        out_shape=jax.ShapeDtypeStruct((m, k), jnp.float32),
    )(ppart, deg_in, deg_out, b)


def _final_layer(ppart, deg_in, b, W):
    def body(p_ref, di_ref, b_ref, w_ref, o_ref):
        p = p_ref[0] + p_ref[1]
        m = p * _norm(di_ref[...])
        o_ref[...] = jnp.dot(m, w_ref[...], precision=lax.Precision.HIGHEST,
                             preferred_element_type=jnp.float32) + b_ref[...]

    _, m, k = ppart.shape
    n = W.shape[1]
    return pl.pallas_call(
        body,
        grid=(m // _BLK,),
        in_specs=[
            pl.BlockSpec((NC, _BLK, k), lambda i: (0, i, 0)),
            pl.BlockSpec((_BLK, 128), lambda i: (i, 0)),
            pl.BlockSpec((1, n), lambda i: (0, 0)),
            pl.BlockSpec((k, n), lambda i: (0, 0)),
        ],
        out_specs=pl.BlockSpec((_BLK, n), lambda i: (i, 0)),
        out_shape=jax.ShapeDtypeStruct((m, n), jnp.float32),
    )(ppart, deg_in, b, W)


def kernel(x, edge_index, W0, b0, W1, b1, W2, b2):
    assert x.shape == (N, 128) and edge_index.shape == (2, E)
    src2d = edge_index[0].astype(jnp.int32).reshape(ROWS, KP)
    dst2d = edge_index[1].astype(jnp.int32).reshape(ROWS, KP)

    C = W2.shape[1]
    W2p = jnp.pad(W2, ((0, 0), (0, 64 - C)))
    b2p = jnp.pad(b2, (0, 64 - C))

    xp = jnp.pad(x, ((0, NP - N), (0, 0)))

    zeros128 = jnp.zeros((NP, 128), jnp.float32)
    ones128 = jnp.ones((KP, 128), jnp.float32)

    ei2d = jnp.concatenate([src2d, dst2d], axis=0)
    degs = _degrees(ei2d, zeros128, ones128)
    z0 = _matmul(xp, W0)
    deg_out = degs[0]
    deg_in = degs[1]

    g0 = _scale_rows(z0, deg_out)
    p0 = _propagate(g0, src2d, dst2d, zeros128)
    g1 = _fused_layer(p0, deg_in, deg_out, b0.reshape(1, -1), W1)
    p1 = _propagate(g1, src2d, dst2d, zeros128)
    g2 = _act_scale(p1, deg_in, deg_out, b1.reshape(1, -1))
    p2 = _propagate(g2, src2d, dst2d, zeros128)
    out = _final_layer(p2, deg_in, b2p.reshape(1, -1), W2p)
    return out[:N, :C]


# speedup vs baseline: 8.7920x; 1.1015x over previous
import functools

import jax
import jax.numpy as jnp
from jax import lax
from jax.experimental import pallas as pl
from jax.experimental.pallas import tpu as pltpu
from jax.experimental.pallas import tpu_sc as plsc

N = 10000
E = 320000
NP = 10112
NZ = NP // 16
NC = 2
NS = 16
KP = 125
EP = 320000
ROWS = EP // KP

_MESH = plsc.VectorSubcoreMesh(core_axis_name="c", subcore_axis_name="s")


def _degrees(ei2d, zeros128, ones128):
    rpw = ROWS // NS
    FIRE = 8

    @functools.partial(
        pl.kernel,
        out_type=jax.ShapeDtypeStruct((NC, NP, 128), jnp.float32),
        mesh=_MESH,
        scratch_types=[
            pltpu.VMEM((rpw, KP), jnp.int32),
            pltpu.VMEM((KP, 128), jnp.float32),
            pltpu.VMEM_SHARED((NP, 128), jnp.float32),
            pltpu.SemaphoreType.DMA,
        ],
    )
    def deg_kernel(ei_hbm, z_hbm, o_hbm, out_hbm, idx_v, ones_v, acc, sem):
        c = lax.axis_index("c")
        s = lax.axis_index("s")
        pltpu.sync_copy(o_hbm, ones_v)
        pltpu.sync_copy(z_hbm.at[pl.ds(s * NZ, NZ)], acc.at[pl.ds(s * NZ, NZ)])
        pltpu.sync_copy(ei_hbm.at[pl.ds(c * ROWS + s * rpw, rpw)], idx_v)
        plsc.subcore_barrier()

        @pl.loop(0, rpw, step=FIRE)
        def _(i):
            for b in range(FIRE):
                pltpu.async_copy(ones_v, acc.at[idx_v.at[i + b]], sem, add=True)
            for b in range(FIRE):
                pltpu.make_async_copy(o_hbm, ones_v, sem).wait()

        plsc.subcore_barrier()
        pltpu.async_copy(
            acc.at[pl.ds(s * NZ, NZ)], out_hbm.at[c].at[pl.ds(s * NZ, NZ)], sem
        ).wait()

    return deg_kernel(ei2d, zeros128, ones128)


def _propagate(g, src2d, dst2d, zeros128):
    rpw = ROWS // (NC * NS)
    rps = 40
    SEG = rpw // rps

    @functools.partial(
        pl.kernel,
        out_type=jax.ShapeDtypeStruct((NC, NP, 128), jnp.float32),
        mesh=_MESH,
        scratch_types=[
            pltpu.VMEM((rps, KP), jnp.int32),
            pltpu.VMEM((rps, KP), jnp.int32),
            pltpu.VMEM((KP, 128), jnp.float32),
            pltpu.VMEM((KP, 128), jnp.float32),
            pltpu.VMEM_SHARED((NP, 128), jnp.float32),
            pltpu.SemaphoreType.DMA,
            pltpu.SemaphoreType.DMA,
        ],
    )
    def prop_kernel(g_hbm, s_hbm, d_hbm, z_hbm, out_hbm,
                    sidx, didx, rows0, rows1, acc, sem0, sem1):
        c = lax.axis_index("c")
        s = lax.axis_index("s")
        r0 = (c * NS + s) * rpw
        pltpu.sync_copy(z_hbm.at[pl.ds(s * NZ, NZ)], acc.at[pl.ds(s * NZ, NZ)])
        plsc.subcore_barrier()

        rows = (rows0, rows1)
        sems = (sem0, sem1)

        @pl.loop(0, SEG)
        def _(seg):
            rseg = r0 + seg * rps
            pltpu.sync_copy(s_hbm.at[pl.ds(rseg, rps)], sidx)
            pltpu.sync_copy(d_hbm.at[pl.ds(rseg, rps)], didx)
            pltpu.async_copy(g_hbm.at[sidx.at[0]], rows0, sem0)
            pltpu.async_copy(g_hbm.at[sidx.at[1]], rows1, sem1)

            @pl.loop(0, rps, step=2)
            def _(i):
                for b in range(2):
                    j = i + b
                    pltpu.make_async_copy(g_hbm.at[sidx.at[j]], rows[b], sems[b]).wait()
                    pltpu.sync_copy(rows[b], acc.at[didx.at[j]], add=True)

                    @pl.when(j + 2 < rps)
                    def _():
                        pltpu.async_copy(g_hbm.at[sidx.at[j + 2]], rows[b], sems[b])

        plsc.subcore_barrier()
        pltpu.async_copy(
            acc.at[pl.ds(s * NZ, NZ)], out_hbm.at[c].at[pl.ds(s * NZ, NZ)], sem0
        ).wait()

    return prop_kernel(g, src2d, dst2d, zeros128)


_BLK = 632


def _norm(deg_blk):
    return lax.rsqrt(jnp.clip(deg_blk[:, :1], 1.0, None))


def _matmul(x, W):
    def body(x_ref, w_ref, o_ref):
        o_ref[...] = jnp.dot(x_ref[...], w_ref[...], precision=lax.Precision.HIGHEST,
                             preferred_element_type=jnp.float32)

    m, k = x.shape
    n = W.shape[1]
    return pl.pallas_call(
        body,
        grid=(m // _BLK,),
        in_specs=[
            pl.BlockSpec((_BLK, k), lambda i: (i, 0)),
            pl.BlockSpec((k, n), lambda i: (0, 0)),
        ],
        out_specs=pl.BlockSpec((_BLK, n), lambda i: (i, 0)),
        out_shape=jax.ShapeDtypeStruct((m, n), jnp.float32),
    )(x, W)


def _scale_rows(z, deg):
    def body(z_ref, d_ref, o_ref):
        o_ref[...] = z_ref[...] * _norm(d_ref[...])

    m, n = z.shape
    return pl.pallas_call(
        body,
        grid=(m // _BLK,),
        in_specs=[
            pl.BlockSpec((_BLK, n), lambda i: (i, 0)),
            pl.BlockSpec((_BLK, 128), lambda i: (i, 0)),
        ],
        out_specs=pl.BlockSpec((_BLK, n), lambda i: (i, 0)),
        out_shape=jax.ShapeDtypeStruct((m, n), jnp.float32),
    )(z, deg)


def _fused_layer(ppart, deg_in, deg_out, b, W):
    def body(p_ref, di_ref, do_ref, b_ref, w_ref, o_ref):
        p = p_ref[0] + p_ref[1]
        h = jax.nn.relu(p * _norm(di_ref[...]) + b_ref[...])
        o_ref[...] = jnp.dot(h, w_ref[...], precision=lax.Precision.HIGHEST,
                             preferred_element_type=jnp.float32) * _norm(do_ref[...])

    _, m, k = ppart.shape
    n = W.shape[1]
    return pl.pallas_call(
        body,
        grid=(m // _BLK,),
        in_specs=[
            pl.BlockSpec((NC, _BLK, k), lambda i: (0, i, 0)),
            pl.BlockSpec((_BLK, 128), lambda i: (i, 0)),
            pl.BlockSpec((_BLK, 128), lambda i: (i, 0)),
            pl.BlockSpec((1, k), lambda i: (0, 0)),
            pl.BlockSpec((k, n), lambda i: (0, 0)),
        ],
        out_specs=pl.BlockSpec((_BLK, n), lambda i: (i, 0)),
        out_shape=jax.ShapeDtypeStruct((m, n), jnp.float32),
    )(ppart, deg_in, deg_out, b, W)


def _act_scale(ppart, deg_in, deg_out, b):
    def body(p_ref, di_ref, do_ref, b_ref, o_ref):
        p = p_ref[0] + p_ref[1]
        h = jax.nn.relu(p * _norm(di_ref[...]) + b_ref[...])
        o_ref[...] = h * _norm(do_ref[...])

    _, m, k = ppart.shape
    return pl.pallas_call(
        body,
        grid=(m // _BLK,),
        in_specs=[
            pl.BlockSpec((NC, _BLK, k), lambda i: (0, i, 0)),
            pl.BlockSpec((_BLK, 128), lambda i: (i, 0)),
            pl.BlockSpec((_BLK, 128), lambda i: (i, 0)),
            pl.BlockSpec((1, k), lambda i: (0, 0)),
        ],
        out_specs=pl.BlockSpec((_BLK, k), lambda i: (i, 0)),
        out_shape=jax.ShapeDtypeStruct((m, k), jnp.float32),
    )(ppart, deg_in, deg_out, b)


def _final_layer(ppart, deg_in, b, W):
    def body(p_ref, di_ref, b_ref, w_ref, o_ref):
        p = p_ref[0] + p_ref[1]
        m = p * _norm(di_ref[...])
        o_ref[...] = jnp.dot(m, w_ref[...], precision=lax.Precision.HIGHEST,
                             preferred_element_type=jnp.float32) + b_ref[...]

    _, m, k = ppart.shape
    n = W.shape[1]
    return pl.pallas_call(
        body,
        grid=(m // _BLK,),
        in_specs=[
            pl.BlockSpec((NC, _BLK, k), lambda i: (0, i, 0)),
            pl.BlockSpec((_BLK, 128), lambda i: (i, 0)),
            pl.BlockSpec((1, n), lambda i: (0, 0)),
            pl.BlockSpec((k, n), lambda i: (0, 0)),
        ],
        out_specs=pl.BlockSpec((_BLK, n), lambda i: (i, 0)),
        out_shape=jax.ShapeDtypeStruct((m, n), jnp.float32),
    )(ppart, deg_in, b, W)


def kernel(x, edge_index, W0, b0, W1, b1, W2, b2):
    assert x.shape == (N, 128) and edge_index.shape == (2, E)
    pad = jnp.full((EP - E,), N, jnp.int32)
    src2d = jnp.concatenate(
        [edge_index[0].astype(jnp.int32), pad]).reshape(ROWS, KP)
    dst2d = jnp.concatenate(
        [edge_index[1].astype(jnp.int32), pad]).reshape(ROWS, KP)

    C = W2.shape[1]
    W2p = jnp.pad(W2, ((0, 0), (0, 64 - C)))
    b2p = jnp.pad(b2, (0, 64 - C))

    xp = jnp.pad(x, ((0, NP - N), (0, 0)))

    zeros128 = jnp.zeros((NP, 128), jnp.float32)
    ones128 = jnp.ones((KP, 128), jnp.float32)

    ei2d = jnp.concatenate([src2d, dst2d], axis=0)
    degs = _degrees(ei2d, zeros128, ones128)
    z0 = _matmul(xp, W0)
    deg_out = degs[0]
    deg_in = degs[1]

    g0 = _scale_rows(z0, deg_out)
    p0 = _propagate(g0, src2d, dst2d, zeros128)
    g1 = _fused_layer(p0, deg_in, deg_out, b0.reshape(1, -1), W1)
    p1 = _propagate(g1, src2d, dst2d, zeros128)
    g2 = _act_scale(p1, deg_in, deg_out, b1.reshape(1, -1))
    p2 = _propagate(g2, src2d, dst2d, zeros128)
    out = _final_layer(p2, deg_in, b2p.reshape(1, -1), W2p)
    return out[:N, :C]
